# Initial kernel scaffold; baseline (speedup 1.0000x reference)
import jax
import jax.numpy as jnp
from jax.experimental import pallas as pl


def kernel(atomic_numbers, pos, edge_index, edge_offsets, cell, three_body_indices, total_num_edges, total_num_angles, total_num_atoms, embedding, enc_W, enc_b, blk_Wg, blk_bg, blk_W2, blk_W3, blk_We, blk_be, blk_We0, blk_Wa, blk_ba, blk_Wa0, en_W1, en_b1, en_W2, en_b2, en_W3, en_b3, scale, shift):
    raise NotImplementedError("write your pallas kernel here")


# speedup vs baseline: 1.3405x; 1.3405x over previous
import functools

import jax
import jax.numpy as jnp
from jax.experimental import pallas as pl
from jax.experimental.pallas import tpu as pltpu

N = 10000
E = 160000
A = 400000
F = 128
NB = 2
MAXN = 4
MAXL = 4
D3 = MAXL * MAXN
NUM_ELEM = 108
CUTOFF = 5.0
TB_CUTOFF = 4.0


def _swish(v):
    return v * jax.nn.sigmoid(v)


def _cutoff(r, rc):
    x = r / rc
    return jnp.where(r < rc, 1.0 - 6.0 * x**5 + 15.0 * x**4 - 10.0 * x**3, 0.0)


def _bessel(r, rc, nmax):
    n = jnp.arange(1, nmax + 1, dtype=jnp.float32)
    rs = jnp.clip(r, 1e-6, None)[:, None]
    return jnp.sqrt(2.0 / rc) * jnp.sin(n * jnp.pi * rs / rc) / rs


def _legendre(x):
    return jnp.stack(
        [jnp.ones_like(x), x, 0.5 * (3.0 * x**2 - 1.0), 0.5 * (5.0 * x**3 - 3.0 * x)],
        axis=-1,
    )


def _gather_rows(table, idx):
    return table[idx]


def _scatter_add_rows(data, idx, nrows):
    return jax.ops.segment_sum(data, idx, num_segments=nrows)


def kernel(atomic_numbers, pos, edge_index, edge_offsets, cell, three_body_indices,
           total_num_edges, total_num_angles, total_num_atoms, embedding, enc_W,
           enc_b, blk_Wg, blk_bg, blk_W2, blk_W3, blk_We, blk_be, blk_We0, blk_Wa,
           blk_ba, blk_Wa0, en_W1, en_b1, en_W2, en_b2, en_W3, en_b3, scale, shift):
    src, dst = edge_index[0], edge_index[1]
    ij = three_body_indices[:, 0]
    ik = three_body_indices[:, 1]

    eoff = edge_offsets @ cell[0]
    p = _gather_rows(pos, src)
    q = _gather_rows(pos, dst)
    edge_vec = q - p + eoff
    edge_dist = jnp.sqrt(jnp.sum(edge_vec * edge_vec, axis=1))
    ef0 = _bessel(edge_dist, CUTOFF, MAXN) * _cutoff(edge_dist, CUTOFF)[:, None]
    c3 = _cutoff(edge_dist, TB_CUTOFF)
    G = jnp.concatenate(
        [edge_vec, edge_dist[:, None], ef0, c3[:, None],
         jnp.zeros((E, 7), jnp.float32)], axis=1)

    Gij = _gather_rows(G, ij)
    Gik = _gather_rows(G, ik)
    vij, dij = Gij[:, :3], Gij[:, 3]
    vik, nik = Gik[:, :3], Gik[:, 3]
    dotp = jnp.sum(vij * vik, axis=1)
    cosang = dotp / (dij * nik)
    eps = float(jnp.finfo(jnp.float32).eps)
    cosang = jnp.clip(cosang, -1.0 + eps, 1.0 - eps)
    rad = _bessel(nik, TB_CUTOFF, MAXN) * _cutoff(nik, TB_CUTOFF)[:, None]
    ang = _legendre(cosang)
    fc3 = _cutoff(dij, TB_CUTOFF) * _cutoff(nik, TB_CUTOFF)
    W3A = (rad[:, None, :] * ang[:, :, None]).reshape(A, D3) * fc3[:, None]

    atom_feat = _gather_rows(embedding, atomic_numbers)
    ef = _swish(G[:, 4:8] @ enc_W + enc_b)

    for b in range(NB):
        gate_k = jax.nn.sigmoid(atom_feat @ blk_Wg[b] + blk_bg[b])
        ge = _gather_rows(gate_k, dst)
        ga = _gather_rows(ge, ik)
        agg = _scatter_add_rows(W3A * ga, ij, E)
        vi = _gather_rows(atom_feat, src)
        vj = _gather_rows(atom_feat, dst)
        ef = ef + _swish(agg @ blk_W2[b]) * jax.nn.sigmoid(agg @ blk_W3[b])
        We = blk_We[b]
        ze = vi @ We[:F] + vj @ We[F:2 * F] + ef @ We[2 * F:] + blk_be[b]
        ef = ef + _swish(ze) * (G[:, 4:8] @ blk_We0[b])
        Wa = blk_Wa[b]
        za = vi @ Wa[:F] + vj @ Wa[F:2 * F] + ef @ Wa[2 * F:] + blk_ba[b]
        msg = _swish(za) * (G[:, 4:8] @ blk_Wa0[b])
        atom_feat = atom_feat + _scatter_add_rows(msg, dst, N)

    h = _swish(atom_feat @ en_W1 + en_b1)
    h = _swish(h @ en_W2 + en_b2)
    e_atom = (h @ en_W3 + en_b3)[:, 0]
    ss = _gather_rows(jnp.stack([scale, shift], axis=1), atomic_numbers)
    e_atom = e_atom * ss[:, 0] + ss[:, 1]
    return jnp.sum(e_atom)[None]


# speedup vs baseline: 2.1206x; 1.5819x over previous
import functools

import jax
import jax.numpy as jnp
from jax import lax
from jax.experimental import pallas as pl
from jax.experimental.pallas import tpu as pltpu
from jax.experimental.pallas import tpu_sc as plsc

N = 10000
E = 160000
A = 400000
F = 128
NB = 2
MAXN = 4
MAXL = 4
D3 = MAXL * MAXN
NUM_ELEM = 108
CUTOFF = 5.0
TB_CUTOFF = 4.0

NC = 2
NS = 16
NW = NC * NS

EP = 163840
AP = 409600
NP = 10240


def _swish(v):
    return v * jax.nn.sigmoid(v)


def _cutoff(r, rc):
    x = r / rc
    return jnp.where(r < rc, 1.0 - 6.0 * x**5 + 15.0 * x**4 - 10.0 * x**3, 0.0)


def _bessel(r, rc, nmax):
    n = jnp.arange(1, nmax + 1, dtype=jnp.float32)
    rs = jnp.clip(r, 1e-6, None)[:, None]
    return jnp.sqrt(2.0 / rc) * jnp.sin(n * jnp.pi * rs / rc) / rs


def _legendre(x):
    return jnp.stack(
        [jnp.ones_like(x), x, 0.5 * (3.0 * x**2 - 1.0), 0.5 * (5.0 * x**3 - 3.0 * x)],
        axis=-1,
    )


def _pad_rows(x, rows, val=0):
    if x.shape[0] == rows:
        return x
    cfg = [(0, rows - x.shape[0])] + [(0, 0)] * (x.ndim - 1)
    return jnp.pad(x, cfg, constant_values=val)


def _pad_cols(x, cols):
    if x.shape[1] == cols:
        return x
    return jnp.pad(x, [(0, 0), (0, cols - x.shape[1])])


_MESH = plsc.VectorSubcoreMesh(
    core_axis_name="c", subcore_axis_name="s", num_cores=NC, num_subcores=NS)
_SC_PARAMS = pltpu.CompilerParams(use_tc_tiling_on_sc=False)


@functools.partial(jax.jit, static_argnames=("chunk",))
def _sc_gather(table, idx, chunk=128):
    B = idx.shape[0]
    V, D = table.shape
    per_w = B // NW
    nchunks = per_w // chunk
    assert per_w % chunk == 0 and B % NW == 0

    @functools.partial(
        pl.kernel,
        out_type=jax.ShapeDtypeStruct((B, D), jnp.float32),
        mesh=_MESH,
        compiler_params=_SC_PARAMS,
        scratch_types=[
            pltpu.VMEM((per_w,), jnp.int32),
            pltpu.VMEM((chunk, D), jnp.float32),
            pltpu.SemaphoreType.DMA,
        ],
    )
    def k(table_hbm, idx_hbm, out_hbm, idx_v, rows_v, sem):
        wid = lax.axis_index("s") * NC + lax.axis_index("c")
        base = wid * per_w
        pltpu.sync_copy(idx_hbm.at[pl.ds(base, per_w)], idx_v)

        def body(i, _):
            pltpu.async_copy(
                table_hbm.at[idx_v.at[pl.ds(i * chunk, chunk)]], rows_v, sem
            ).wait()
            pltpu.sync_copy(rows_v, out_hbm.at[pl.ds(base + i * chunk, chunk)])
            return ()

        lax.fori_loop(0, nchunks, body, (), unroll=False)

    return k(table, idx)


@functools.partial(jax.jit, static_argnames=("chunk",))
def _sc_scatter_add(data, idx, init, chunk=128):
    B, D = data.shape
    V = init.shape[0]
    Vh = V // NC
    per_t = B // NS
    nchunks = per_t // chunk
    assert per_t % chunk == 0 and V % 256 == 0
    rows_t = Vh // NS

    @functools.partial(
        pl.kernel,
        out_type=jax.ShapeDtypeStruct((V, D), jnp.float32),
        mesh=_MESH,
        compiler_params=_SC_PARAMS,
        scratch_types=[
            pltpu.VMEM_SHARED((Vh + 8, D), jnp.float32),
            pltpu.VMEM((chunk,), jnp.int32),
            pltpu.VMEM((chunk,), jnp.int32),
            pltpu.VMEM((chunk, D), jnp.float32),
        ],
    )
    def k(data_hbm, idx_hbm, init_hbm, out_hbm, acc_sh, idx_v, adj_v, dat_v):
        cid = lax.axis_index("c")
        sid = lax.axis_index("s")
        lo = cid * Vh
        pltpu.sync_copy(init_hbm.at[pl.ds(lo + sid * rows_t, rows_t)],
                        acc_sh.at[pl.ds(sid * rows_t, rows_t)])

        @pl.when(sid == 0)
        def _():
            pltpu.sync_copy(init_hbm.at[pl.ds(0, 8)], acc_sh.at[pl.ds(Vh, 8)])

        plsc.subcore_barrier()

        def body(i, _):
            off = sid * per_t + i * chunk
            pltpu.sync_copy(idx_hbm.at[pl.ds(off, chunk)], idx_v)
            pltpu.sync_copy(data_hbm.at[pl.ds(off, chunk)], dat_v)
            for j in range(chunk // 16):
                v = idx_v[pl.ds(j * 16, 16)]
                inr = (v >= lo) & (v < lo + Vh)
                adj_v[pl.ds(j * 16, 16)] = jnp.where(inr, v - lo, Vh)
            pltpu.sync_copy(dat_v, acc_sh.at[adj_v], add=True)
            return ()

        lax.fori_loop(0, nchunks, body, (), unroll=False)
        plsc.subcore_barrier()
        pltpu.sync_copy(acc_sh.at[pl.ds(sid * rows_t, rows_t)],
                        out_hbm.at[pl.ds(lo + sid * rows_t, rows_t)])

    return k(data, idx, init)


def kernel(atomic_numbers, pos, edge_index, edge_offsets, cell, three_body_indices,
           total_num_edges, total_num_angles, total_num_atoms, embedding, enc_W,
           enc_b, blk_Wg, blk_bg, blk_W2, blk_W3, blk_We, blk_be, blk_We0, blk_Wa,
           blk_ba, blk_Wa0, en_W1, en_b1, en_W2, en_b2, en_W3, en_b3, scale, shift):
    src, dst = edge_index[0], edge_index[1]
    ij = three_body_indices[:, 0]
    ik = three_body_indices[:, 1]

    src_g = _pad_rows(src, EP)
    dst_g = _pad_rows(dst, EP)
    ij_g = _pad_rows(ij, AP)
    ik_g = _pad_rows(ik, AP)
    ij_s = _pad_rows(ij, AP, val=E)
    dst_s = _pad_rows(dst, EP, val=N)

    pos16 = _pad_cols(pos, 16)

    p = _sc_gather(pos16, src_g)
    q = _sc_gather(pos16, dst_g)
    eoff = _pad_rows(edge_offsets, EP) @ cell[0]
    edge_vec = q[:, :3] - p[:, :3] + eoff
    edge_dist = jnp.sqrt(jnp.sum(edge_vec * edge_vec, axis=1))
    ef0 = _bessel(edge_dist, CUTOFF, MAXN) * _cutoff(edge_dist, CUTOFF)[:, None]
    c3 = _cutoff(edge_dist, TB_CUTOFF)
    G = jnp.concatenate(
        [edge_vec, edge_dist[:, None], ef0, c3[:, None],
         jnp.zeros((EP, 7), jnp.float32)], axis=1)

    Gij = _sc_gather(G, ij_g)
    Gik = _sc_gather(G, ik_g)
    vij, dij = Gij[:, :3], Gij[:, 3]
    vik, nik = Gik[:, :3], Gik[:, 3]
    cosang = jnp.sum(vij * vik, axis=1) / (dij * nik)
    eps = float(jnp.finfo(jnp.float32).eps)
    cosang = jnp.clip(cosang, -1.0 + eps, 1.0 - eps)
    rad = _bessel(nik, TB_CUTOFF, MAXN) * _cutoff(nik, TB_CUTOFF)[:, None]
    ang = _legendre(cosang)
    fc3 = _cutoff(dij, TB_CUTOFF) * _cutoff(nik, TB_CUTOFF)
    W3A = (rad[:, None, :] * ang[:, :, None]).reshape(AP, D3) * fc3[:, None]

    atom_feat = _sc_gather(embedding, _pad_rows(atomic_numbers, NP), chunk=64)
    ef = _swish(G[:, 4:8] @ enc_W + enc_b)

    zero16 = jnp.zeros((EP, D3), jnp.float32)
    for b in range(NB):
        gate_k = jax.nn.sigmoid(atom_feat @ blk_Wg[b] + blk_bg[b])
        ge = _sc_gather(gate_k, dst_g)
        ga = _sc_gather(ge, ik_g)
        agg = _sc_scatter_add(W3A * ga, ij_s, zero16)
        vi = _sc_gather(atom_feat, src_g)
        vj = _sc_gather(atom_feat, dst_g)
        ef = ef + _swish(agg @ blk_W2[b]) * jax.nn.sigmoid(agg @ blk_W3[b])
        We = blk_We[b]
        ze = vi @ We[:F] + vj @ We[F:2 * F] + ef @ We[2 * F:] + blk_be[b]
        ef = ef + _swish(ze) * (G[:, 4:8] @ blk_We0[b])
        Wa = blk_Wa[b]
        za = vi @ Wa[:F] + vj @ Wa[F:2 * F] + ef @ Wa[2 * F:] + blk_ba[b]
        msg = _swish(za) * (G[:, 4:8] @ blk_Wa0[b])
        atom_feat = _sc_scatter_add(msg, dst_s, atom_feat)

    h = _swish(atom_feat @ en_W1 + en_b1)
    h = _swish(h @ en_W2 + en_b2)
    e_atom = (h @ en_W3 + en_b3)[:, 0]
    ss_tab = _pad_rows(_pad_cols(jnp.stack([scale, shift], axis=1), 16), 112)
    ss = _sc_gather(ss_tab, _pad_rows(atomic_numbers, NP, val=108), chunk=64)
    e_atom = e_atom * ss[:, 0] + ss[:, 1]
    return jnp.sum(e_atom)[None]


# speedup vs baseline: 2.3129x; 1.0907x over previous
import functools

import jax
import jax.numpy as jnp
from jax import lax
from jax.experimental import pallas as pl
from jax.experimental.pallas import tpu as pltpu
from jax.experimental.pallas import tpu_sc as plsc

N = 10000
E = 160000
A = 400000
F = 128
NB = 2
MAXN = 4
MAXL = 4
D3 = MAXL * MAXN
NUM_ELEM = 108
CUTOFF = 5.0
TB_CUTOFF = 4.0

NC = 2
NS = 16
NW = NC * NS

EP = 163840
AP = 409600
NP = 10240

KB = 4


def _swish(v):
    return v * jax.nn.sigmoid(v)


def _cutoff(r, rc):
    x = r / rc
    return jnp.where(r < rc, 1.0 - 6.0 * x**5 + 15.0 * x**4 - 10.0 * x**3, 0.0)


def _bessel(r, rc, nmax):
    n = jnp.arange(1, nmax + 1, dtype=jnp.float32)
    rs = jnp.clip(r, 1e-6, None)[:, None]
    return jnp.sqrt(2.0 / rc) * jnp.sin(n * jnp.pi * rs / rc) / rs


def _legendre(x):
    return jnp.stack(
        [jnp.ones_like(x), x, 0.5 * (3.0 * x**2 - 1.0), 0.5 * (5.0 * x**3 - 3.0 * x)],
        axis=-1,
    )


def _pad_rows(x, rows, val=0):
    if x.shape[0] == rows:
        return x
    cfg = [(0, rows - x.shape[0])] + [(0, 0)] * (x.ndim - 1)
    return jnp.pad(x, cfg, constant_values=val)


def _pad_cols(x, cols):
    if x.shape[1] == cols:
        return x
    return jnp.pad(x, [(0, 0), (0, cols - x.shape[1])])


_MESH = plsc.VectorSubcoreMesh(
    core_axis_name="c", subcore_axis_name="s", num_cores=NC, num_subcores=NS)
_SC_PARAMS = pltpu.CompilerParams(use_tc_tiling_on_sc=False)


@functools.partial(jax.jit, static_argnames=("chunk",))
def _sc_gather(table, idx, chunk=128):
    B = idx.shape[0]
    V, D = table.shape
    per_w = B // NW
    nchunks = per_w // chunk
    assert B % NW == 0 and per_w % chunk == 0 and nchunks % KB == 0 and nchunks >= KB

    @functools.partial(
        pl.kernel,
        out_type=jax.ShapeDtypeStruct((B, D), jnp.float32),
        mesh=_MESH,
        compiler_params=_SC_PARAMS,
        scratch_types=[
            pltpu.VMEM((per_w,), jnp.int32),
            pltpu.VMEM((KB, chunk, D), jnp.float32),
            pltpu.SemaphoreType.DMA((KB,)),
            pltpu.SemaphoreType.DMA((KB,)),
        ],
    )
    def k(table_hbm, idx_hbm, out_hbm, idx_v, bufs, gsem, ssem):
        wid = lax.axis_index("s") * NC + lax.axis_index("c")
        base = wid * per_w
        pltpu.sync_copy(idx_hbm.at[pl.ds(base, per_w)], idx_v)

        def issue(i, s):
            pltpu.async_copy(
                table_hbm.at[idx_v.at[pl.ds(i * chunk, chunk)]],
                bufs.at[s], gsem.at[s])

        for s in range(2):
            issue(s, s)

        @pl.loop(0, nchunks, step=KB)
        def _(i0):
            for j in range(KB):
                i = i0 + j
                jl = (j + 2) % KB

                @pl.when(i + 2 < nchunks)
                def _():
                    @pl.when(i + 2 >= KB)
                    def _():
                        pltpu.make_async_copy(
                            bufs.at[jl], out_hbm.at[pl.ds(base, chunk)],
                            ssem.at[jl]).wait()
                    issue(i + 2, jl)

                pltpu.make_async_copy(
                    out_hbm.at[pl.ds(base, chunk)], bufs.at[j], gsem.at[j]).wait()
                pltpu.async_copy(
                    bufs.at[j], out_hbm.at[pl.ds(base + i * chunk, chunk)],
                    ssem.at[j])

        for j in range(KB):
            pltpu.make_async_copy(
                bufs.at[j], out_hbm.at[pl.ds(base, chunk)], ssem.at[j]).wait()

    return k(table, idx)


@jax.jit
def _sc_angle_agg(gate, nidx, w, ij, init):
    chunk = 128
    KA = 5
    Vh = EP // NC
    per_t = AP // NS
    nchunks = per_t // chunk
    rows_t = Vh // NS
    assert nchunks % KA == 0 and nchunks >= KA

    @functools.partial(
        pl.kernel,
        out_type=jax.ShapeDtypeStruct((EP, D3), jnp.float32),
        mesh=_MESH,
        compiler_params=_SC_PARAMS,
        scratch_types=[
            pltpu.VMEM_SHARED((Vh + 8, D3), jnp.float32),
            pltpu.VMEM((KA, chunk), jnp.int32),
            pltpu.VMEM((KA, chunk), jnp.int32),
            pltpu.VMEM((KA, chunk, D3), jnp.float32),
            pltpu.VMEM((KA, chunk, D3), jnp.float32),
            pltpu.VMEM((KA, chunk), jnp.int32),
            pltpu.SemaphoreType.DMA((KA,)),
            pltpu.SemaphoreType.DMA((KA,)),
            pltpu.SemaphoreType.DMA((KA,)),
            pltpu.SemaphoreType.DMA((KA,)),
            pltpu.SemaphoreType.DMA((KA,)),
        ],
    )
    def k(gate_hbm, nidx_hbm, w_hbm, ij_hbm, init_hbm, out_hbm,
          acc_sh, nbuf, ibuf, gbuf, wbuf, adj, nsem, isem, gsem, wsem, ssem):
        cid = lax.axis_index("c")
        sid = lax.axis_index("s")
        lo = cid * Vh
        pltpu.sync_copy(init_hbm.at[pl.ds(lo + sid * rows_t, rows_t)],
                        acc_sh.at[pl.ds(sid * rows_t, rows_t)])

        @pl.when(sid == 0)
        def _():
            pltpu.sync_copy(init_hbm.at[pl.ds(0, 8)], acc_sh.at[pl.ds(Vh, 8)])

        base = sid * per_t
        plsc.subcore_barrier()

        def issue_idx(i, s):
            pltpu.async_copy(nidx_hbm.at[pl.ds(base + i * chunk, chunk)],
                             nbuf.at[s], nsem.at[s])
            pltpu.async_copy(ij_hbm.at[pl.ds(base + i * chunk, chunk)],
                             ibuf.at[s], isem.at[s])

        def issue_gather(i, s):
            pltpu.make_async_copy(
                nidx_hbm.at[pl.ds(base, chunk)], nbuf.at[s], nsem.at[s]).wait()
            pltpu.async_copy(gate_hbm.at[nbuf.at[s]], gbuf.at[s], gsem.at[s])
            pltpu.async_copy(w_hbm.at[pl.ds(base + i * chunk, chunk)],
                             wbuf.at[s], wsem.at[s])

        for c in range(4):
            issue_idx(c, c)
        for c in range(2):
            issue_gather(c, c)

        @pl.loop(0, nchunks, step=KA)
        def _(i0):
            for j in range(KA):
                i = i0 + j
                sg = (j + 2) % KA
                sa = (j + 4) % KA

                @pl.when(i + 4 < nchunks)
                def _():
                    issue_idx(i + 4, sa)

                @pl.when(i + 2 < nchunks)
                def _():
                    @pl.when(i + 2 >= KA)
                    def _():
                        pltpu.make_async_copy(
                            out_hbm.at[pl.ds(0, chunk)], gbuf.at[sg],
                            ssem.at[sg]).wait()
                    issue_gather(i + 2, sg)

                pltpu.make_async_copy(
                    out_hbm.at[pl.ds(0, chunk)], gbuf.at[j], gsem.at[j]).wait()
                pltpu.make_async_copy(
                    out_hbm.at[pl.ds(0, chunk)], wbuf.at[j], wsem.at[j]).wait()
                pltpu.make_async_copy(
                    ij_hbm.at[pl.ds(base, chunk)], ibuf.at[j], isem.at[j]).wait()
                for r in range(chunk):
                    gbuf[j, r, :] = gbuf[j, r, :] * wbuf[j, r, :]
                for t in range(chunk // 16):
                    v = ibuf[j, pl.ds(t * 16, 16)]
                    inr = (v >= lo) & (v < lo + Vh)
                    adj[j, pl.ds(t * 16, 16)] = jnp.where(inr, v - lo, Vh)
                pltpu.async_copy(gbuf.at[j], acc_sh.at[adj.at[j]], ssem.at[j],
                                 add=True)

        for j in range(KA):
            pltpu.make_async_copy(
                out_hbm.at[pl.ds(0, chunk)], gbuf.at[j], ssem.at[j]).wait()
        plsc.subcore_barrier()
        pltpu.sync_copy(acc_sh.at[pl.ds(sid * rows_t, rows_t)],
                        out_hbm.at[pl.ds(lo + sid * rows_t, rows_t)])

    return k(gate, nidx, w, ij, init)


@jax.jit
def _sc_scatter_add(data, idx, init):
    chunk = 128
    B, D = data.shape
    V = init.shape[0]
    Vh = V // NC
    per_t = B // NS
    nchunks = per_t // chunk
    rows_t = Vh // NS
    assert per_t % chunk == 0 and nchunks % KB == 0 and V % (NC * NS) == 0

    @functools.partial(
        pl.kernel,
        out_type=jax.ShapeDtypeStruct((V, D), jnp.float32),
        mesh=_MESH,
        compiler_params=_SC_PARAMS,
        scratch_types=[
            pltpu.VMEM_SHARED((Vh + 8, D), jnp.float32),
            pltpu.VMEM((per_t,), jnp.int32),
            pltpu.VMEM((KB, chunk, D), jnp.float32),
            pltpu.VMEM((KB, chunk), jnp.int32),
            pltpu.SemaphoreType.DMA((KB,)),
            pltpu.SemaphoreType.DMA((KB,)),
        ],
    )
    def k(data_hbm, idx_hbm, init_hbm, out_hbm,
          acc_sh, idx_v, dbuf, adj, dsem, ssem):
        cid = lax.axis_index("c")
        sid = lax.axis_index("s")
        lo = cid * Vh
        pltpu.sync_copy(init_hbm.at[pl.ds(lo + sid * rows_t, rows_t)],
                        acc_sh.at[pl.ds(sid * rows_t, rows_t)])

        @pl.when(sid == 0)
        def _():
            pltpu.sync_copy(init_hbm.at[pl.ds(0, 8)], acc_sh.at[pl.ds(Vh, 8)])

        base = sid * per_t
        pltpu.sync_copy(idx_hbm.at[pl.ds(base, per_t)], idx_v)
        plsc.subcore_barrier()

        def issue(i, s):
            pltpu.async_copy(
                data_hbm.at[pl.ds(base + i * chunk, chunk)], dbuf.at[s],
                dsem.at[s])

        for s in range(2):
            issue(s, s)

        @pl.loop(0, nchunks, step=KB)
        def _(i0):
            for j in range(KB):
                i = i0 + j
                jl = (j + 2) % KB

                @pl.when(i + 2 < nchunks)
                def _():
                    @pl.when(i + 2 >= KB)
                    def _():
                        pltpu.make_async_copy(
                            out_hbm.at[pl.ds(0, chunk)], dbuf.at[jl],
                            ssem.at[jl]).wait()
                    issue(i + 2, jl)

                pltpu.make_async_copy(
                    out_hbm.at[pl.ds(0, chunk)], dbuf.at[j], dsem.at[j]).wait()
                for t in range(chunk // 16):
                    v = idx_v[pl.ds(i * chunk + t * 16, 16)]
                    inr = (v >= lo) & (v < lo + Vh)
                    adj[j, pl.ds(t * 16, 16)] = jnp.where(inr, v - lo, Vh)
                pltpu.async_copy(dbuf.at[j], acc_sh.at[adj.at[j]], ssem.at[j],
                                 add=True)

        for j in range(KB):
            pltpu.make_async_copy(
                out_hbm.at[pl.ds(0, chunk)], dbuf.at[j], ssem.at[j]).wait()
        plsc.subcore_barrier()
        pltpu.sync_copy(acc_sh.at[pl.ds(sid * rows_t, rows_t)],
                        out_hbm.at[pl.ds(lo + sid * rows_t, rows_t)])

    return k(data, idx, init)


def kernel(atomic_numbers, pos, edge_index, edge_offsets, cell, three_body_indices,
           total_num_edges, total_num_angles, total_num_atoms, embedding, enc_W,
           enc_b, blk_Wg, blk_bg, blk_W2, blk_W3, blk_We, blk_be, blk_We0, blk_Wa,
           blk_ba, blk_Wa0, en_W1, en_b1, en_W2, en_b2, en_W3, en_b3, scale, shift):
    src, dst = edge_index[0], edge_index[1]
    ij = three_body_indices[:, 0]
    ik = three_body_indices[:, 1]

    src_g = _pad_rows(src, EP)
    dst_g = _pad_rows(dst, EP)
    ij_g = _pad_rows(ij, AP)
    ik_g = _pad_rows(ik, AP)
    ij_s = _pad_rows(ij, AP, val=E)
    dst_s = _pad_rows(dst, EP, val=N)

    pos16 = _pad_cols(pos, 16)

    p = _sc_gather(pos16, src_g)
    q = _sc_gather(pos16, dst_g)
    eoff = _pad_rows(edge_offsets, EP) @ cell[0]
    edge_vec = q[:, :3] - p[:, :3] + eoff
    edge_dist = jnp.sqrt(jnp.sum(edge_vec * edge_vec, axis=1))
    ef0 = _bessel(edge_dist, CUTOFF, MAXN) * _cutoff(edge_dist, CUTOFF)[:, None]
    c3 = _cutoff(edge_dist, TB_CUTOFF)
    G = jnp.concatenate(
        [edge_vec, edge_dist[:, None], ef0, c3[:, None],
         lax.bitcast_convert_type(dst_g, jnp.float32)[:, None],
         jnp.zeros((EP, 6), jnp.float32)], axis=1)

    Gij = _sc_gather(G, ij_g)
    Gik = _sc_gather(G, ik_g)
    nidx = lax.bitcast_convert_type(Gik[:, 9], jnp.int32)
    vij, dij = Gij[:, :3], Gij[:, 3]
    vik, nik = Gik[:, :3], Gik[:, 3]
    cosang = jnp.sum(vij * vik, axis=1) / (dij * nik)
    eps = float(jnp.finfo(jnp.float32).eps)
    cosang = jnp.clip(cosang, -1.0 + eps, 1.0 - eps)
    rad = _bessel(nik, TB_CUTOFF, MAXN) * _cutoff(nik, TB_CUTOFF)[:, None]
    ang = _legendre(cosang)
    fc3 = _cutoff(dij, TB_CUTOFF) * _cutoff(nik, TB_CUTOFF)
    W3A = (rad[:, None, :] * ang[:, :, None]).reshape(AP, D3) * fc3[:, None]

    atom_feat = _sc_gather(embedding, _pad_rows(atomic_numbers, NP), chunk=80)
    ef = _swish(G[:, 4:8] @ enc_W + enc_b)

    zero16 = jnp.zeros((EP, D3), jnp.float32)
    for b in range(NB):
        gate_k = jax.nn.sigmoid(atom_feat @ blk_Wg[b] + blk_bg[b])
        agg = _sc_angle_agg(gate_k, nidx, W3A, ij_s, zero16)
        vi = _sc_gather(atom_feat, src_g)
        vj = _sc_gather(atom_feat, dst_g)
        ef = ef + _swish(agg @ blk_W2[b]) * jax.nn.sigmoid(agg @ blk_W3[b])
        We = blk_We[b]
        ze = vi @ We[:F] + vj @ We[F:2 * F] + ef @ We[2 * F:] + blk_be[b]
        ef = ef + _swish(ze) * (G[:, 4:8] @ blk_We0[b])
        Wa = blk_Wa[b]
        za = vi @ Wa[:F] + vj @ Wa[F:2 * F] + ef @ Wa[2 * F:] + blk_ba[b]
        msg = _swish(za) * (G[:, 4:8] @ blk_Wa0[b])
        atom_feat = _sc_scatter_add(msg, dst_s, atom_feat)

    h = _swish(atom_feat @ en_W1 + en_b1)
    h = _swish(h @ en_W2 + en_b2)
    e_atom = (h @ en_W3 + en_b3)[:, 0]
    ss_tab = _pad_rows(_pad_cols(jnp.stack([scale, shift], axis=1), 16), 112)
    ss = _sc_gather(ss_tab, _pad_rows(atomic_numbers, NP, val=108), chunk=80)
    e_atom = e_atom * ss[:, 0] + ss[:, 1]
    return jnp.sum(e_atom)[None]


# speedup vs baseline: 4.8516x; 2.0976x over previous
import functools

import jax
import jax.numpy as jnp
from jax import lax
from jax.experimental import pallas as pl
from jax.experimental.pallas import tpu as pltpu
from jax.experimental.pallas import tpu_sc as plsc

N = 10000
E = 160000
A = 400000
F = 128
NB = 2
MAXN = 4
MAXL = 4
D3 = MAXL * MAXN
NUM_ELEM = 108
CUTOFF = 5.0
TB_CUTOFF = 4.0

NC = 2
NS = 16
NW = NC * NS

EP = 163840
AP = 409600
NP = 10240

KB = 4


def _swish(v):
    return v * jax.nn.sigmoid(v)


def _cutoff(r, rc):
    x = r / rc
    return jnp.where(r < rc, 1.0 - 6.0 * x**5 + 15.0 * x**4 - 10.0 * x**3, 0.0)


def _bessel(r, rc, nmax):
    n = jnp.arange(1, nmax + 1, dtype=jnp.float32)
    rs = jnp.clip(r, 1e-6, None)[:, None]
    return jnp.sqrt(2.0 / rc) * jnp.sin(n * jnp.pi * rs / rc) / rs


def _legendre(x):
    return jnp.stack(
        [jnp.ones_like(x), x, 0.5 * (3.0 * x**2 - 1.0), 0.5 * (5.0 * x**3 - 3.0 * x)],
        axis=-1,
    )


def _pad_rows(x, rows, val=0):
    if x.shape[0] == rows:
        return x
    cfg = [(0, rows - x.shape[0])] + [(0, 0)] * (x.ndim - 1)
    return jnp.pad(x, cfg, constant_values=val)


def _pad_cols(x, cols):
    if x.shape[1] == cols:
        return x
    return jnp.pad(x, [(0, 0), (0, cols - x.shape[1])])


_MESH = plsc.VectorSubcoreMesh(
    core_axis_name="c", subcore_axis_name="s", num_cores=NC, num_subcores=NS)
_SC_PARAMS = pltpu.CompilerParams(use_tc_tiling_on_sc=False)


@functools.partial(jax.jit, static_argnames=("chunk",))
def _sc_gather(table, idx, chunk=128):
    B = idx.shape[0]
    V, D = table.shape
    per_w = B // NW
    nchunks = per_w // chunk
    assert B % NW == 0 and per_w % chunk == 0 and nchunks % KB == 0 and nchunks >= KB

    @functools.partial(
        pl.kernel,
        out_type=jax.ShapeDtypeStruct((B, D), jnp.float32),
        mesh=_MESH,
        compiler_params=_SC_PARAMS,
        scratch_types=[
            pltpu.VMEM((per_w,), jnp.int32),
            pltpu.VMEM((KB, chunk, D), jnp.float32),
            pltpu.SemaphoreType.DMA((KB,)),
            pltpu.SemaphoreType.DMA((KB,)),
        ],
    )
    def k(table_hbm, idx_hbm, out_hbm, idx_v, bufs, gsem, ssem):
        wid = lax.axis_index("s") * NC + lax.axis_index("c")
        base = wid * per_w
        pltpu.sync_copy(idx_hbm.at[pl.ds(base, per_w)], idx_v)

        def issue(i, s):
            pltpu.async_copy(
                table_hbm.at[idx_v.at[pl.ds(i * chunk, chunk)]],
                bufs.at[s], gsem.at[s])

        for s in range(2):
            issue(s, s)

        @pl.loop(0, nchunks, step=KB)
        def _(i0):
            for j in range(KB):
                i = i0 + j
                jl = (j + 2) % KB

                @pl.when(i + 2 < nchunks)
                def _():
                    @pl.when(i + 2 >= KB)
                    def _():
                        pltpu.make_async_copy(
                            bufs.at[jl], out_hbm.at[pl.ds(base, chunk)],
                            ssem.at[jl]).wait()
                    issue(i + 2, jl)

                pltpu.make_async_copy(
                    out_hbm.at[pl.ds(base, chunk)], bufs.at[j], gsem.at[j]).wait()
                pltpu.async_copy(
                    bufs.at[j], out_hbm.at[pl.ds(base + i * chunk, chunk)],
                    ssem.at[j])

        for j in range(KB):
            pltpu.make_async_copy(
                bufs.at[j], out_hbm.at[pl.ds(base, chunk)], ssem.at[j]).wait()

    return k(table, idx)


@jax.jit
def _sc_angle_agg(gate, nidx, w, ij, init):
    chunk = 128
    KA = 5
    Vh = EP // NC
    per_t = AP // NS
    nchunks = per_t // chunk
    rows_t = Vh // NS
    assert nchunks % KA == 0 and nchunks >= KA

    @functools.partial(
        pl.kernel,
        out_type=jax.ShapeDtypeStruct((EP, D3), jnp.float32),
        mesh=_MESH,
        compiler_params=_SC_PARAMS,
        scratch_types=[
            pltpu.VMEM_SHARED((Vh + 8, D3), jnp.float32),
            pltpu.VMEM((KA, chunk), jnp.int32),
            pltpu.VMEM((KA, chunk), jnp.int32),
            pltpu.VMEM((KA, chunk, D3), jnp.float32),
            pltpu.VMEM((KA, chunk, D3), jnp.float32),
            pltpu.VMEM((KA, chunk), jnp.int32),
            pltpu.SemaphoreType.DMA((KA,)),
            pltpu.SemaphoreType.DMA((KA,)),
            pltpu.SemaphoreType.DMA((KA,)),
            pltpu.SemaphoreType.DMA((KA,)),
            pltpu.SemaphoreType.DMA((KA,)),
        ],
    )
    def k(gate_hbm, nidx_hbm, w_hbm, ij_hbm, init_hbm, out_hbm,
          acc_sh, nbuf, ibuf, gbuf, wbuf, adj, nsem, isem, gsem, wsem, ssem):
        cid = lax.axis_index("c")
        sid = lax.axis_index("s")
        lo = cid * Vh
        pltpu.sync_copy(init_hbm.at[pl.ds(lo + sid * rows_t, rows_t)],
                        acc_sh.at[pl.ds(sid * rows_t, rows_t)])

        @pl.when(sid == 0)
        def _():
            pltpu.sync_copy(init_hbm.at[pl.ds(0, 8)], acc_sh.at[pl.ds(Vh, 8)])

        base = sid * per_t
        plsc.subcore_barrier()

        def issue_idx(i, s):
            pltpu.async_copy(nidx_hbm.at[pl.ds(base + i * chunk, chunk)],
                             nbuf.at[s], nsem.at[s])
            pltpu.async_copy(ij_hbm.at[pl.ds(base + i * chunk, chunk)],
                             ibuf.at[s], isem.at[s])

        def issue_gather(i, s):
            pltpu.make_async_copy(
                nidx_hbm.at[pl.ds(base, chunk)], nbuf.at[s], nsem.at[s]).wait()
            pltpu.async_copy(gate_hbm.at[nbuf.at[s]], gbuf.at[s], gsem.at[s])
            pltpu.async_copy(w_hbm.at[pl.ds(base + i * chunk, chunk)],
                             wbuf.at[s], wsem.at[s])

        for c in range(4):
            issue_idx(c, c)
        for c in range(2):
            issue_gather(c, c)

        @pl.loop(0, nchunks, step=KA)
        def _(i0):
            for j in range(KA):
                i = i0 + j
                sg = (j + 2) % KA
                sa = (j + 4) % KA

                @pl.when(i + 4 < nchunks)
                def _():
                    issue_idx(i + 4, sa)

                @pl.when(i + 2 < nchunks)
                def _():
                    @pl.when(i + 2 >= KA)
                    def _():
                        pltpu.make_async_copy(
                            out_hbm.at[pl.ds(0, chunk)], gbuf.at[sg],
                            ssem.at[sg]).wait()
                    issue_gather(i + 2, sg)

                pltpu.make_async_copy(
                    out_hbm.at[pl.ds(0, chunk)], gbuf.at[j], gsem.at[j]).wait()
                pltpu.make_async_copy(
                    out_hbm.at[pl.ds(0, chunk)], wbuf.at[j], wsem.at[j]).wait()
                pltpu.make_async_copy(
                    ij_hbm.at[pl.ds(base, chunk)], ibuf.at[j], isem.at[j]).wait()
                for r in range(chunk):
                    gbuf[j, r, :] = gbuf[j, r, :] * wbuf[j, r, :]
                for t in range(chunk // 16):
                    v = ibuf[j, pl.ds(t * 16, 16)]
                    inr = (v >= lo) & (v < lo + Vh)
                    adj[j, pl.ds(t * 16, 16)] = jnp.where(inr, v - lo, Vh)
                pltpu.async_copy(gbuf.at[j], acc_sh.at[adj.at[j]], ssem.at[j],
                                 add=True)

        for j in range(KA):
            pltpu.make_async_copy(
                out_hbm.at[pl.ds(0, chunk)], gbuf.at[j], ssem.at[j]).wait()
        plsc.subcore_barrier()
        pltpu.sync_copy(acc_sh.at[pl.ds(sid * rows_t, rows_t)],
                        out_hbm.at[pl.ds(lo + sid * rows_t, rows_t)])

    return k(gate, nidx, w, ij, init)


@jax.jit
def _sc_scatter_add(data, idx, init):
    chunk = 128
    B, D = data.shape
    V = init.shape[0]
    Vh = V // NC
    per_t = B // NS
    nchunks = per_t // chunk
    rows_t = Vh // NS
    assert per_t % chunk == 0 and nchunks % KB == 0 and V % (NC * NS) == 0

    @functools.partial(
        pl.kernel,
        out_type=jax.ShapeDtypeStruct((V, D), jnp.float32),
        mesh=_MESH,
        compiler_params=_SC_PARAMS,
        scratch_types=[
            pltpu.VMEM_SHARED((Vh + 8, D), jnp.float32),
            pltpu.VMEM((per_t,), jnp.int32),
            pltpu.VMEM((KB, chunk, D), jnp.float32),
            pltpu.VMEM((KB, chunk), jnp.int32),
            pltpu.SemaphoreType.DMA((KB,)),
            pltpu.SemaphoreType.DMA((KB,)),
        ],
    )
    def k(data_hbm, idx_hbm, init_hbm, out_hbm,
          acc_sh, idx_v, dbuf, adj, dsem, ssem):
        cid = lax.axis_index("c")
        sid = lax.axis_index("s")
        lo = cid * Vh
        pltpu.sync_copy(init_hbm.at[pl.ds(lo + sid * rows_t, rows_t)],
                        acc_sh.at[pl.ds(sid * rows_t, rows_t)])

        @pl.when(sid == 0)
        def _():
            pltpu.sync_copy(init_hbm.at[pl.ds(0, 8)], acc_sh.at[pl.ds(Vh, 8)])

        base = sid * per_t
        pltpu.sync_copy(idx_hbm.at[pl.ds(base, per_t)], idx_v)
        plsc.subcore_barrier()

        def issue(i, s):
            pltpu.async_copy(
                data_hbm.at[pl.ds(base + i * chunk, chunk)], dbuf.at[s],
                dsem.at[s])

        for s in range(2):
            issue(s, s)

        @pl.loop(0, nchunks, step=KB)
        def _(i0):
            for j in range(KB):
                i = i0 + j
                jl = (j + 2) % KB

                @pl.when(i + 2 < nchunks)
                def _():
                    @pl.when(i + 2 >= KB)
                    def _():
                        pltpu.make_async_copy(
                            out_hbm.at[pl.ds(0, chunk)], dbuf.at[jl],
                            ssem.at[jl]).wait()
                    issue(i + 2, jl)

                pltpu.make_async_copy(
                    out_hbm.at[pl.ds(0, chunk)], dbuf.at[j], dsem.at[j]).wait()
                for t in range(chunk // 16):
                    v = idx_v[pl.ds(i * chunk + t * 16, 16)]
                    inr = (v >= lo) & (v < lo + Vh)
                    adj[j, pl.ds(t * 16, 16)] = jnp.where(inr, v - lo, Vh)
                pltpu.async_copy(dbuf.at[j], acc_sh.at[adj.at[j]], ssem.at[j],
                                 add=True)

        for j in range(KB):
            pltpu.make_async_copy(
                out_hbm.at[pl.ds(0, chunk)], dbuf.at[j], ssem.at[j]).wait()
        plsc.subcore_barrier()
        pltpu.sync_copy(acc_sh.at[pl.ds(sid * rows_t, rows_t)],
                        out_hbm.at[pl.ds(lo + sid * rows_t, rows_t)])

    return k(data, idx, init)


_BE = 512


def _lane16(shape):
    return lax.broadcasted_iota(jnp.int32, shape, 1) % 16


def _cut_tc(d, rc):
    x = d / rc
    return jnp.where(d < rc, 1.0 - 6.0 * x**5 + 15.0 * x**4 - 10.0 * x**3, 0.0)


def _geom_body(p_ref, q_ref, o_ref, db_ref, mc_ref, ms_ref, g_ref):
    V = q_ref[...] - p_ref[...] + jnp.dot(
        o_ref[...], mc_ref[...], preferred_element_type=jnp.float32)
    d2 = jnp.dot(V * V, ms_ref[...], preferred_element_type=jnp.float32)
    d = jnp.sqrt(d2)
    lane = _lane16(V.shape)
    nl = (lane - 3).astype(jnp.float32)
    b = (jnp.sqrt(2.0 / CUTOFF) * jnp.sin(nl * (jnp.pi / CUTOFF) * d)
         / jnp.maximum(d, 1e-6)) * _cut_tc(d, CUTOFF)
    out = jnp.where(lane < 3, V, 0.0)
    out = jnp.where(lane == 3, d, out)
    out = jnp.where((lane >= 4) & (lane < 8), b, out)
    out = jnp.where(lane == 8, _cut_tc(d, TB_CUTOFF), out)
    out = jnp.where(lane == 9, db_ref[...], out)
    g_ref[...] = out


@jax.jit
def _tc_geom(pv, qv, ov, dbv, mcell, msum3):
    R = pv.shape[0]
    grid = R // _BE
    bs = lambda: pl.BlockSpec((_BE, 128), lambda i: (i, 0))
    ws = pl.BlockSpec((128, 128), lambda i: (0, 0))
    return pl.pallas_call(
        _geom_body,
        out_shape=jax.ShapeDtypeStruct((R, 128), jnp.float32),
        grid=(grid,),
        in_specs=[bs(), bs(), bs(), bs(), ws, ws],
        out_specs=bs(),
    )(pv, qv, ov, dbv, mcell, msum3)


def _ang_body(gi_ref, gk_ref, ms_ref, mr_ref, w_ref):
    gi = gi_ref[...]
    gk = gk_ref[...]
    dot = jnp.dot(gi * gk, ms_ref[...], preferred_element_type=jnp.float32)
    q2 = jnp.dot(gi * gi, ms_ref[...], preferred_element_type=jnp.float32)
    r2 = jnp.dot(gk * gk, ms_ref[...], preferred_element_type=jnp.float32)
    dij = jnp.dot(gi, mr_ref[...], preferred_element_type=jnp.float32)
    nik = jnp.dot(gk, mr_ref[...], preferred_element_type=jnp.float32)
    eps = float(jnp.finfo(jnp.float32).eps)
    cos = jnp.clip(dot * lax.rsqrt(q2 * r2), -1.0 + eps, 1.0 - eps)
    lane = _lane16(gi.shape)
    nb = (lane % 4 + 1).astype(jnp.float32)
    radl = (jnp.sqrt(2.0 / TB_CUTOFF) * jnp.sin(nb * (jnp.pi / TB_CUTOFF) * nik)
            / jnp.maximum(nik, 1e-6)) * _cut_tc(nik, TB_CUTOFF)
    l_idx = lane // 4
    angl = jnp.where(l_idx == 0, 1.0, cos)
    angl = jnp.where(l_idx == 2, 0.5 * (3.0 * cos * cos - 1.0), angl)
    angl = jnp.where(l_idx == 3, 0.5 * (5.0 * cos**3 - 3.0 * cos), angl)
    fc3 = _cut_tc(dij, TB_CUTOFF) * _cut_tc(nik, TB_CUTOFF)
    w_ref[...] = radl * angl * fc3


@jax.jit
def _tc_ang(giv, gkv, msum3, mrep3):
    R = giv.shape[0]
    grid = R // _BE
    bs = lambda: pl.BlockSpec((_BE, 128), lambda i: (i, 0))
    ws = pl.BlockSpec((128, 128), lambda i: (0, 0))
    return pl.pallas_call(
        _ang_body,
        out_shape=jax.ShapeDtypeStruct((R, 128), jnp.float32),
        grid=(grid,),
        in_specs=[bs(), bs(), ws, ws],
        out_specs=bs(),
    )(giv, gkv, msum3, mrep3)


def _gate_body(af_ref, wg_ref, bg_ref, o_ref):
    o_ref[...] = jax.nn.sigmoid(
        jnp.dot(af_ref[...], wg_ref[...], preferred_element_type=jnp.float32)
        + bg_ref[...])


@jax.jit
def _tc_gate(af, Wg, bg):
    grid = NP // _BE
    return pl.pallas_call(
        _gate_body,
        out_shape=jax.ShapeDtypeStruct((NP, D3), jnp.float32),
        grid=(grid,),
        in_specs=[pl.BlockSpec((_BE, 128), lambda i: (i, 0)),
                  pl.BlockSpec((128, D3), lambda i: (0, 0)),
                  pl.BlockSpec((1, D3), lambda i: (0, 0))],
        out_specs=pl.BlockSpec((_BE, D3), lambda i: (i, 0)),
    )(af, Wg, bg.reshape(1, D3))


def _ef0_body(g_ref, w_ref, b_ref, o_ref):
    o_ref[...] = _swish(
        jnp.dot(g_ref[...], w_ref[...], preferred_element_type=jnp.float32)
        + b_ref[...])


@jax.jit
def _tc_ef0(G, encWp, encb):
    grid = EP // _BE
    return pl.pallas_call(
        _ef0_body,
        out_shape=jax.ShapeDtypeStruct((EP, F), jnp.float32),
        grid=(grid,),
        in_specs=[pl.BlockSpec((_BE, 16), lambda i: (i, 0)),
                  pl.BlockSpec((16, F), lambda i: (0, 0)),
                  pl.BlockSpec((1, F), lambda i: (0, 0))],
        out_specs=pl.BlockSpec((_BE, F), lambda i: (i, 0)),
    )(G, encWp, encb.reshape(1, F))


def _edge_body(vi_ref, vj_ref, ef_ref, g_ref, agg_ref, w2_ref, w3_ref,
               we1_ref, we2_ref, we3_ref, be_ref, we0_ref,
               wa1_ref, wa2_ref, wa3_ref, ba_ref, wa0_ref,
               ef2_ref, msg_ref):
    dotf = lambda a, b: jnp.dot(a, b[...], preferred_element_type=jnp.float32)
    agg = agg_ref[...]
    efu = ef_ref[...] + _swish(dotf(agg, w2_ref)) * jax.nn.sigmoid(
        dotf(agg, w3_ref))
    vi = vi_ref[...]
    vj = vj_ref[...]
    g = g_ref[...]
    ze = dotf(vi, we1_ref) + dotf(vj, we2_ref) + dotf(efu, we3_ref) + be_ref[...]
    ef2 = efu + _swish(ze) * dotf(g, we0_ref)
    za = dotf(vi, wa1_ref) + dotf(vj, wa2_ref) + dotf(ef2, wa3_ref) + ba_ref[...]
    ef2_ref[...] = ef2
    msg_ref[...] = _swish(za) * dotf(g, wa0_ref)


@jax.jit
def _tc_edge(vi, vj, ef, G, agg, W2, W3, We1, We2, We3, be, We0p,
             Wa1, Wa2, Wa3, ba, Wa0p):
    grid = EP // _BE
    bF = pl.BlockSpec((_BE, F), lambda i: (i, 0))
    b16 = pl.BlockSpec((_BE, 16), lambda i: (i, 0))
    w16 = pl.BlockSpec((16, F), lambda i: (0, 0))
    wF = pl.BlockSpec((F, F), lambda i: (0, 0))
    w1 = pl.BlockSpec((1, F), lambda i: (0, 0))
    return pl.pallas_call(
        _edge_body,
        out_shape=[jax.ShapeDtypeStruct((EP, F), jnp.float32),
                   jax.ShapeDtypeStruct((EP, F), jnp.float32)],
        grid=(grid,),
        in_specs=[bF, bF, bF, b16, b16, w16, w16,
                  wF, wF, wF, w1, w16, wF, wF, wF, w1, w16],
        out_specs=[bF, bF],
    )(vi, vj, ef, G, agg, W2, W3, We1, We2, We3, be.reshape(1, F), We0p,
      Wa1, Wa2, Wa3, ba.reshape(1, F), Wa0p)


def _readout_body(af_ref, ss_ref, w1_ref, b1_ref, w2_ref, b2_ref, w3_ref,
                  b3_ref, o_ref):
    dotf = lambda a, b: jnp.dot(a, b[...], preferred_element_type=jnp.float32)
    h = _swish(dotf(af_ref[...], w1_ref) + b1_ref[...])
    h = _swish(dotf(h, w2_ref) + b2_ref[...])
    z = dotf(h, w3_ref)
    e = (z[:, 0:1] + b3_ref[...]) * ss_ref[:, 0:1] + ss_ref[:, 1:2]

    @pl.when(pl.program_id(0) == 0)
    def _():
        o_ref[...] = jnp.zeros_like(o_ref)

    o_ref[...] += jnp.sum(e, keepdims=True)


@jax.jit
def _tc_readout(af, ss, W1, b1, W2, b2, W3p, b3):
    grid = NP // _BE
    wF = pl.BlockSpec((F, F), lambda i: (0, 0))
    w1 = pl.BlockSpec((1, F), lambda i: (0, 0))
    out = pl.pallas_call(
        _readout_body,
        out_shape=jax.ShapeDtypeStruct((1, 1), jnp.float32),
        grid=(grid,),
        in_specs=[pl.BlockSpec((_BE, F), lambda i: (i, 0)),
                  pl.BlockSpec((_BE, 16), lambda i: (i, 0)),
                  wF, w1, wF, w1, wF,
                  pl.BlockSpec((1, 1), lambda i: (0, 0))],
        out_specs=pl.BlockSpec((1, 1), lambda i: (0, 0)),
    )(af, ss, W1, b1.reshape(1, F), W2, b2.reshape(1, F), W3p,
      b3.reshape(1, 1))
    return out[:, 0]


def kernel(atomic_numbers, pos, edge_index, edge_offsets, cell, three_body_indices,
           total_num_edges, total_num_angles, total_num_atoms, embedding, enc_W,
           enc_b, blk_Wg, blk_bg, blk_W2, blk_W3, blk_We, blk_be, blk_We0, blk_Wa,
           blk_ba, blk_Wa0, en_W1, en_b1, en_W2, en_b2, en_W3, en_b3, scale, shift):
    src, dst = edge_index[0], edge_index[1]
    ij = three_body_indices[:, 0]
    ik = three_body_indices[:, 1]

    src_g = _pad_rows(src, EP)
    dst_g = _pad_rows(dst, EP)
    ij_g = _pad_rows(ij, AP)
    ik_g = _pad_rows(ik, AP)
    ij_s = _pad_rows(ij, AP, val=E)
    dst_s = _pad_rows(dst, EP, val=N)

    pos16 = _pad_cols(pos, 16)
    eye8 = jnp.eye(8, dtype=jnp.float32)
    mcell = jnp.kron(eye8, jnp.pad(cell[0], [(0, 13), (0, 13)]))
    msum3 = jnp.kron(eye8, jnp.pad(jnp.ones((3, 16), jnp.float32),
                                   [(0, 13), (0, 0)]))
    mrep3 = jnp.kron(eye8, jnp.pad(jnp.ones((1, 16), jnp.float32),
                                   [(3, 12), (0, 0)]))

    p = _sc_gather(pos16, src_g)
    q = _sc_gather(pos16, dst_g)
    ov = _pad_cols(_pad_rows(edge_offsets, EP), 16).reshape(EP // 8, 128)
    dbv = jnp.pad(lax.bitcast_convert_type(dst_g, jnp.float32)[:, None],
                  [(0, 0), (9, 6)]).reshape(EP // 8, 128)
    Gv = _tc_geom(p.reshape(EP // 8, 128), q.reshape(EP // 8, 128), ov, dbv,
                  mcell, msum3)
    G = Gv.reshape(EP, 16)

    Gij = _sc_gather(G, ij_g)
    Gik = _sc_gather(G, ik_g)
    nidx = lax.bitcast_convert_type(Gik[:, 9], jnp.int32)
    W3A = _tc_ang(Gij.reshape(AP // 8, 128), Gik.reshape(AP // 8, 128),
                  msum3, mrep3).reshape(AP, 16)

    atom_feat = _sc_gather(embedding, _pad_rows(atomic_numbers, NP), chunk=80)
    ef = _tc_ef0(G, jnp.pad(enc_W, [(4, 8), (0, 0)]), enc_b)

    zero16 = jnp.zeros((EP, D3), jnp.float32)
    for b in range(NB):
        gate_k = _tc_gate(atom_feat, blk_Wg[b], blk_bg[b])
        agg = _sc_angle_agg(gate_k, nidx, W3A, ij_s, zero16)
        vi = _sc_gather(atom_feat, src_g)
        vj = _sc_gather(atom_feat, dst_g)
        We = blk_We[b]
        Wa = blk_Wa[b]
        ef, msg = _tc_edge(
            vi, vj, ef, G, agg, blk_W2[b], blk_W3[b],
            We[:F], We[F:2 * F], We[2 * F:], blk_be[b],
            jnp.pad(blk_We0[b], [(4, 8), (0, 0)]),
            Wa[:F], Wa[F:2 * F], Wa[2 * F:], blk_ba[b],
            jnp.pad(blk_Wa0[b], [(4, 8), (0, 0)]))
        atom_feat = _sc_scatter_add(msg, dst_s, atom_feat)

    ss_tab = _pad_rows(_pad_cols(jnp.stack([scale, shift], axis=1), 16), 112)
    ss = _sc_gather(ss_tab, _pad_rows(atomic_numbers, NP, val=108), chunk=80)
    return _tc_readout(atom_feat, ss, en_W1, en_b1, en_W2, en_b2,
                       jnp.pad(en_W3, [(0, 0), (0, 127)]), en_b3)
